# baseline (speedup 1.0000x reference)
import functools

import jax
import jax.numpy as jnp
from jax import lax
from jax.experimental import pallas as pl
from jax.experimental.pallas import tpu as pltpu, tpu_sc as plsc

_ALPHA = 1.0
_BETA = 3.0

_N = 2_000_000
_TABLE = 1024
_LUT = 1216
_CHUNK = 16000
_NCHUNKS = _N // _CHUNK
_UNROLL = 10
_NACC = 4

_NC, _NS, _LANES = 2, 16, 16
_NW = _NC * _NS
_TRIPS = -(-_NCHUNKS // _NW)
_PAIRS = -(-_TRIPS // 2)


def _sc_body(table_hbm, fec_hbm, pk_hbm, out_hbm,
             table_v, fec_v, lut_v, pk_v, acc_v, sem0, sem1):
    wid = lax.axis_index("s") * _NC + lax.axis_index("c")
    lane = lax.iota(jnp.int32, _LANES)
    sems = (sem0, sem1)

    my_chunks = (_NCHUNKS - 1 - wid) // _NW + 1
    last_ci = my_chunks - 1

    def start1(b, ci):
        g = wid + jnp.minimum(ci, last_ci) * _NW
        pltpu.async_copy(pk_hbm.at[pl.ds(g * _CHUNK, _CHUNK)], pk_v.at[b],
                         sems[b])

    def wait1(b):
        pltpu.make_async_copy(pk_hbm.at[pl.ds(0, _CHUNK)], pk_v.at[b],
                              sems[b]).wait()

    start1(0, 0)

    pltpu.sync_copy(table_hbm, table_v)
    pltpu.sync_copy(fec_hbm, fec_v)

    def lut_body(i, _):
        v = i * _LANES + lane
        lo = jnp.zeros((_LANES,), jnp.int32)
        for s in (1024, 512, 256, 128, 64, 32, 16, 8, 4, 2, 1):
            cand = lo + s
            probe = jnp.minimum(cand, _TABLE) - 1
            t = plsc.load_gather(table_v, [probe])
            ok = (cand <= _TABLE) & (t < v)
            lo = jnp.where(ok, cand, lo)
        idx = jnp.clip(lo - 1, 0, _TABLE - 1)
        lut_v[pl.ds(i * _LANES, _LANES)] = plsc.load_gather(fec_v, [idx])
        return 0

    lax.fori_loop(0, _LUT // _LANES, lut_body, 0)

    zero = jnp.zeros((_LANES,), jnp.float32)
    izero = jnp.zeros((_LANES,), jnp.int32)
    ione = jnp.full((_LANES,), 1, jnp.int32)
    zero4 = (zero, zero, zero, izero)

    fzero = jnp.zeros((_LANES,), jnp.float32)

    def compute(b, ci, accs):
        pkb = pk_v.at[b]

        @plsc.parallel_loop(0, _CHUNK // _LANES, 1, unroll=_UNROLL,
                            carry=accs)
        def vloop(v, a):
            a_sqm, a_squ, a_rat, a_cnt = a
            p = pkb[pl.ds(v * _LANES, _LANES)]
            fs_i = p & 2047
            lp_i = (p >> 11) & 63
            ratio = plsc.load_gather(lut_v, [fs_i])
            fsf = fs_i.astype(jnp.float32)
            lpf = lp_i.astype(jnp.float32)
            d = lpf - ratio * fsf
            sq = d * d
            sqm = jnp.where(p >= (1 << 17), sq, fzero)
            return (a_sqm + sqm, a_squ + (sq - sqm), a_rat + ratio,
                    a_cnt + jnp.minimum(lp_i, ione))

        upd = vloop
        ok = ci < my_chunks
        return jax.tree.map(lambda nw, od: jnp.where(ok, nw, od), upd, accs)

    def pair_body(pi, accs):
        ci0 = pi * 2
        start1(1, ci0 + 1)
        wait1(0)
        accs = compute(0, ci0, accs)
        start1(0, ci0 + 2)
        wait1(1)
        return compute(1, ci0 + 1, accs)

    accs = lax.fori_loop(0, _PAIRS, pair_body, zero4)
    wait1(0)

    for k in range(4):
        tot = accs[k]
        if k == 3:
            tot = tot.astype(jnp.float32)
        acc_v[pl.ds(k * _LANES, _LANES)] = tot
    pltpu.sync_copy(acc_v, out_hbm.at[wid])


_sc_call = functools.partial(
    pl.kernel,
    out_type=jax.ShapeDtypeStruct((_NW, 4 * _LANES), jnp.float32),
    mesh=plsc.VectorSubcoreMesh(core_axis_name="c", subcore_axis_name="s"),
    compiler_params=pltpu.CompilerParams(use_tc_tiling_on_sc=False,
                                         needs_layout_passes=False),
    scratch_types=[
        pltpu.VMEM((_TABLE,), jnp.int32),
        pltpu.VMEM((_TABLE,), jnp.float32),
        pltpu.VMEM((_LUT,), jnp.float32),
        pltpu.VMEM((2, _CHUNK), jnp.int32),
        pltpu.VMEM((4 * _LANES,), jnp.float32),
        pltpu.SemaphoreType.DMA,
        pltpu.SemaphoreType.DMA,
    ],
)(_sc_body)


def kernel(pred_bitrate, pred_fec, fec_level_table, frame_size,
           loss_packets, recovery_status):
    n = frame_size.shape[0]
    packed = (frame_size | (loss_packets << 11)
              | (recovery_status.astype(jnp.int32) << 17))

    parts = _sc_call(fec_level_table, pred_fec, packed)

    sums = parts.reshape(_NW, 4, _LANES).sum(axis=(0, 2))
    s_rec, s_unrec, s_ratio, cnt = sums[0], sums[1], sums[2], sums[3]
    inv_n = jnp.float32(1.0 / n)
    loss_fec_opt = _ALPHA * jnp.sqrt(s_rec) + _BETA * jnp.sqrt(s_unrec)
    loss_reward = pred_bitrate + s_ratio * inv_n
    loss_rate = cnt * inv_n
    return loss_fec_opt + loss_reward + loss_rate * pred_bitrate


# speedup vs baseline: 1.1360x; 1.1360x over previous
import functools

import jax
import jax.numpy as jnp
from jax import lax
from jax.experimental import pallas as pl
from jax.experimental.pallas import tpu as pltpu, tpu_sc as plsc

_ALPHA = 1.0
_BETA = 3.0

_N = 2_000_000
_TABLE = 1024
_LUT = 1216
_CHUNK = 16000
_NCHUNKS = _N // _CHUNK
_UNROLL = 10

_NC, _NS, _LANES = 2, 16, 16
_NW = _NC * _NS
_TRIPS = -(-_NCHUNKS // _NW)
_PAIRS = -(-_TRIPS // 2)


def _sc_body(table_hbm, fec_hbm, pk_hbm, lp_hbm, out_hbm,
             table_v, fec_v, lut_v, lut3_v, pk_v, lp_v, acc_v, sem0, sem1):
    wid = lax.axis_index("s") * _NC + lax.axis_index("c")
    lane = lax.iota(jnp.int32, _LANES)
    sems = (sem0, sem1)

    my_chunks = (_NCHUNKS - 1 - wid) // _NW + 1
    last_ci = my_chunks - 1

    def start2(b, ci):
        g = wid + jnp.minimum(ci, last_ci) * _NW
        pltpu.async_copy(pk_hbm.at[pl.ds(g * _CHUNK, _CHUNK)], pk_v.at[b],
                         sems[b])
        pltpu.async_copy(lp_hbm.at[pl.ds(g * _CHUNK, _CHUNK)], lp_v.at[b],
                         sems[b])

    def wait2(b):
        pltpu.make_async_copy(pk_hbm.at[pl.ds(0, _CHUNK)], pk_v.at[b],
                              sems[b]).wait()
        pltpu.make_async_copy(lp_hbm.at[pl.ds(0, _CHUNK)], lp_v.at[b],
                              sems[b]).wait()

    start2(0, 0)

    pltpu.sync_copy(table_hbm, table_v)
    pltpu.sync_copy(fec_hbm, fec_v)

    def lut_body(i, _):
        v = i * _LANES + lane
        lo = jnp.zeros((_LANES,), jnp.int32)
        for s in (1024, 512, 256, 128, 64, 32, 16, 8, 4, 2, 1):
            cand = lo + s
            probe = jnp.minimum(cand, _TABLE) - 1
            t = plsc.load_gather(table_v, [probe])
            ok = (cand <= _TABLE) & (t < v)
            lo = jnp.where(ok, cand, lo)
        idx = jnp.clip(lo - 1, 0, _TABLE - 1)
        val = plsc.load_gather(fec_v, [idx])
        lut_v[pl.ds(i * _LANES, _LANES)] = val
        lut3_v[pl.ds(i * _LANES, _LANES)] = val * v.astype(jnp.float32)
        return 0

    lax.fori_loop(0, _LUT // _LANES, lut_body, 0)

    zero = jnp.zeros((_LANES,), jnp.float32)
    fone = jnp.full((_LANES,), 1.0, jnp.float32)
    zero4 = (zero, zero, zero, zero)

    def compute(b, ci, accs):
        pkb, lpb = pk_v.at[b], lp_v.at[b]

        @plsc.parallel_loop(0, _CHUNK // _LANES, 1, unroll=_UNROLL,
                            carry=accs)
        def vloop(v, a):
            a_sqm, a_sq, a_rat, a_cnt = a
            p = pkb[pl.ds(v * _LANES, _LANES)]
            lp_i = lpb[pl.ds(v * _LANES, _LANES)]
            fs_i = p & 2047
            fp = plsc.load_gather(lut3_v, [fs_i])
            ratio = plsc.load_gather(lut_v, [fs_i])
            lpf = lp_i.astype(jnp.float32)
            d = lpf - fp
            sq = d * d
            sqm = jnp.where(p >= 2048, sq, zero)
            return (a_sqm + sqm, a_sq + sq, a_rat + ratio,
                    a_cnt + jnp.minimum(lpf, fone))

        upd = vloop
        ok = ci < my_chunks
        return jax.tree.map(lambda nw, od: jnp.where(ok, nw, od), upd, accs)

    def pair_body(pi, accs):
        ci0 = pi * 2
        start2(1, ci0 + 1)
        wait2(0)
        accs = compute(0, ci0, accs)
        start2(0, ci0 + 2)
        wait2(1)
        return compute(1, ci0 + 1, accs)

    accs = lax.fori_loop(0, _PAIRS, pair_body, zero4)
    wait2(0)

    a_sqm, a_sq, a_rat, a_cnt = accs
    acc_v[pl.ds(0, _LANES)] = a_sqm
    acc_v[pl.ds(_LANES, _LANES)] = a_sq - a_sqm
    acc_v[pl.ds(2 * _LANES, _LANES)] = a_rat
    acc_v[pl.ds(3 * _LANES, _LANES)] = a_cnt
    pltpu.sync_copy(acc_v, out_hbm.at[wid])


_sc_call = functools.partial(
    pl.kernel,
    out_type=jax.ShapeDtypeStruct((_NW, 4 * _LANES), jnp.float32),
    mesh=plsc.VectorSubcoreMesh(core_axis_name="c", subcore_axis_name="s"),
    compiler_params=pltpu.CompilerParams(use_tc_tiling_on_sc=False,
                                         needs_layout_passes=False),
    scratch_types=[
        pltpu.VMEM((_TABLE,), jnp.int32),
        pltpu.VMEM((_TABLE,), jnp.float32),
        pltpu.VMEM((_LUT,), jnp.float32),
        pltpu.VMEM((_LUT,), jnp.float32),
        pltpu.VMEM((2, _CHUNK), jnp.int32),
        pltpu.VMEM((2, _CHUNK), jnp.int32),
        pltpu.VMEM((4 * _LANES,), jnp.float32),
        pltpu.SemaphoreType.DMA,
        pltpu.SemaphoreType.DMA,
    ],
)(_sc_body)


def kernel(pred_bitrate, pred_fec, fec_level_table, frame_size,
           loss_packets, recovery_status):
    n = frame_size.shape[0]
    packed = frame_size | (recovery_status.astype(jnp.int32) << 11)

    parts = _sc_call(fec_level_table, pred_fec, packed, loss_packets)

    sums = parts.reshape(_NW, 4, _LANES).sum(axis=(0, 2))
    s_rec, s_unrec, s_ratio, cnt = sums[0], sums[1], sums[2], sums[3]
    inv_n = jnp.float32(1.0 / n)
    loss_fec_opt = _ALPHA * jnp.sqrt(s_rec) + _BETA * jnp.sqrt(s_unrec)
    loss_reward = pred_bitrate + s_ratio * inv_n
    loss_rate = cnt * inv_n
    return loss_fec_opt + loss_reward + loss_rate * pred_bitrate
